# Initial kernel scaffold; baseline (speedup 1.0000x reference)
import functools

import jax
import jax.numpy as jnp
from jax import lax
from jax.experimental import pallas as pl
from jax.experimental.pallas import tpu as pltpu
from jax.experimental.pallas import tpu_sc as plsc

B = 2
C = 128
N = 1728
M = B * N
K = 9
RT = 216
EPS = 1e-5

NC = 2
NS = 16
NW = NC * NS
NPW = M // NW
CH = 12
NCHUNK = NPW // CH
IDXPAD = 112


def _bn_cols(y, g, b):
    m = jnp.mean(y, axis=0, keepdims=True)
    v = jnp.mean((y - m) * (y - m), axis=0, keepdims=True)
    return (y - m) * lax.rsqrt(v + EPS) * g + b


def _stem_body(p_ref, w_ref, b_ref, o_ref):
    o_ref[...] = lax.dot_general(
        p_ref[...], w_ref[...], (((1,), (1,)), ((), ())),
        preferred_element_type=jnp.float32) + b_ref[...]


def _stem_matmul(patches, w2d, b2d):
    return pl.pallas_call(
        _stem_body,
        grid=(M // RT,),
        in_specs=[
            pl.BlockSpec((RT, 2048), lambda i: (i, 0)),
            pl.BlockSpec((C, 2048), lambda i: (0, 0)),
            pl.BlockSpec((1, C), lambda i: (0, 0)),
        ],
        out_specs=pl.BlockSpec((RT, C), lambda i: (i, 0)),
        out_shape=jax.ShapeDtypeStruct((M, C), jnp.float32),
    )(patches, w2d, b2d)


def _bn_relu_body(x_ref, g_ref, b_ref, o_ref):
    o_ref[...] = jnp.maximum(_bn_cols(x_ref[...], g_ref[...], b_ref[...]), 0.0)


def _bn_relu(x, g, b):
    return pl.pallas_call(
        _bn_relu_body,
        out_shape=jax.ShapeDtypeStruct(x.shape, jnp.float32),
    )(x, g.reshape(1, -1), b.reshape(1, -1))


def _fc1_body(x_ref, w_ref, b_ref, g_ref, bb_ref, o_ref):
    y = lax.dot_general(x_ref[...], w_ref[...], (((1,), (1,)), ((), ())),
                        preferred_element_type=jnp.float32) + b_ref[...]
    o_ref[...] = _bn_cols(y, g_ref[...], bb_ref[...])


def _fc1_bn(x, w, b, g, bb):
    return pl.pallas_call(
        _fc1_body,
        out_shape=jax.ShapeDtypeStruct((M, C), jnp.float32),
    )(x, w, b.reshape(1, C), g.reshape(1, C), bb.reshape(1, C))


def _topk_body(fa_ref, ft_ref, o_ref):
    fa = fa_ref[0]
    ft = ft_ref[0]
    b = pl.program_id(0)
    g = lax.dot_general(ft, fa, (((1,), (1,)), ((), ())),
                        preferred_element_type=jnp.float32)
    sqa = jnp.sum(fa * fa, axis=1)[None, :]
    sqt = jnp.sum(ft * ft, axis=1)[:, None]
    dist = sqt - 2.0 * g + sqa
    cols = lax.broadcasted_iota(jnp.int32, (RT, N), 1)
    picks = []
    for _ in range(K):
        m = jnp.min(dist, axis=1, keepdims=True)
        sel = dist == m
        idx = jnp.min(jnp.where(sel, cols, N), axis=1, keepdims=True)
        picks.append(idx)
        dist = jnp.where(cols == idx, jnp.inf, dist)
    o_ref[0] = jnp.concatenate(picks, axis=1) + b * N


def _topk(f3):
    return pl.pallas_call(
        _topk_body,
        grid=(B, N // RT),
        in_specs=[
            pl.BlockSpec((1, N, C), lambda b, r: (b, 0, 0)),
            pl.BlockSpec((1, RT, C), lambda b, r: (b, r, 0)),
        ],
        out_specs=pl.BlockSpec((1, RT, K), lambda b, r: (b, r, 0)),
        out_shape=jax.ShapeDtypeStruct((B, N, K), jnp.int32),
    )(f3, f3)


def _mr_body(f_ref, xj_ref, xin_ref, we_ref, wo_ref, mb_ref, mg_ref, mbb_ref,
             w2_ref, b2_ref, g2_ref, b2b_ref, o_ref):
    h = (lax.dot_general(f_ref[...], we_ref[...], (((1,), (1,)), ((), ())),
                         preferred_element_type=jnp.float32)
         + lax.dot_general(xj_ref[...], wo_ref[...], (((1,), (1,)), ((), ())),
                           preferred_element_type=jnp.float32)
         + mb_ref[...])
    h = _bn_cols(h, mg_ref[...], mbb_ref[...])
    h = jax.nn.gelu(h)
    out = lax.dot_general(h, w2_ref[...], (((1,), (1,)), ((), ())),
                          preferred_element_type=jnp.float32) + b2_ref[...]
    out = _bn_cols(out, g2_ref[...], b2b_ref[...])
    o_ref[...] = jnp.maximum(out + xin_ref[...], 0.0)


def _mr_fc2(f, xj, xin, we, wo, mb, mg, mbb, w2, b2, g2, b2b):
    return pl.pallas_call(
        _mr_body,
        out_shape=jax.ShapeDtypeStruct((M, C), jnp.float32),
    )(f, xj, xin, we, wo, mb.reshape(1, 2 * C), mg.reshape(1, 2 * C),
      mbb.reshape(1, 2 * C), w2, b2.reshape(1, C), g2.reshape(1, C),
      b2b.reshape(1, C))


def _sc_maxrel_body(f_hbm, idx_hbm, out_hbm, idx_v, rows_v, own_v, out_v, sem):
    wid = lax.axis_index("s") * NC + lax.axis_index("c")

    def chunk(ci, carry):
        base = wid * NPW + ci * CH
        pltpu.sync_copy(idx_hbm.at[wid, ci], idx_v)
        pltpu.async_copy(f_hbm.at[idx_v], rows_v, sem).wait()
        pltpu.sync_copy(f_hbm.at[pl.ds(base, CH)], own_v)
        for j in range(CH):
            for t in range(C // 16):
                s = pl.ds(t * 16, 16)
                acc = rows_v[j * K, s]
                for q in range(1, K):
                    acc = jnp.maximum(acc, rows_v[j * K + q, s])
                out_v[j, s] = acc - own_v[j, s]
        pltpu.sync_copy(out_v, out_hbm.at[pl.ds(base, CH)])
        return carry

    lax.fori_loop(0, NCHUNK, chunk, 0)


@functools.partial(
    pl.kernel,
    out_type=jax.ShapeDtypeStruct((M, C), jnp.float32),
    mesh=plsc.VectorSubcoreMesh(core_axis_name="c", subcore_axis_name="s"),
    scratch_types=[
        pltpu.VMEM((IDXPAD,), jnp.int32),
        pltpu.VMEM((IDXPAD, C), jnp.float32),
        pltpu.VMEM((CH, C), jnp.float32),
        pltpu.VMEM((CH, C), jnp.float32),
        pltpu.SemaphoreType.DMA,
    ],
)
def _sc_maxrel(f_hbm, idx_hbm, out_hbm, idx_v, rows_v, own_v, out_v, sem):
    _sc_maxrel_body(f_hbm, idx_hbm, out_hbm, idx_v, rows_v, own_v, out_v, sem)


def _max_relative(f, idx):
    idx_pad = jnp.pad(idx.reshape(NW, NCHUNK, CH * K),
                      ((0, 0), (0, 0), (0, IDXPAD - CH * K)))
    return _sc_maxrel(f, idx_pad)


def kernel(x, stem_w, stem_b, stem_bng, stem_bnb,
           blk0_fc1_w, blk0_fc1_b, blk0_bn1_g, blk0_bn1_b,
           blk0_mr_w, blk0_mr_b, blk0_mrbn_g, blk0_mrbn_b,
           blk0_fc2_w, blk0_fc2_b, blk0_bn2_g, blk0_bn2_b,
           blk1_fc1_w, blk1_fc1_b, blk1_bn1_g, blk1_bn1_b,
           blk1_mr_w, blk1_mr_b, blk1_mrbn_g, blk1_mrbn_b,
           blk1_fc2_w, blk1_fc2_b, blk1_bn2_g, blk1_bn2_b):
    patches = x.reshape(B, 4, 12, 8, 12, 8, 12, 8)
    patches = patches.transpose(0, 2, 4, 6, 1, 3, 5, 7).reshape(M, 4 * 512)
    y = _stem_matmul(patches, stem_w.reshape(C, 4 * 512), stem_b.reshape(1, C))
    xcur = _bn_relu(y, stem_bng, stem_bnb)

    blocks = [
        (blk0_fc1_w, blk0_fc1_b, blk0_bn1_g, blk0_bn1_b, blk0_mr_w, blk0_mr_b,
         blk0_mrbn_g, blk0_mrbn_b, blk0_fc2_w, blk0_fc2_b, blk0_bn2_g, blk0_bn2_b),
        (blk1_fc1_w, blk1_fc1_b, blk1_bn1_g, blk1_bn1_b, blk1_mr_w, blk1_mr_b,
         blk1_mrbn_g, blk1_mrbn_b, blk1_fc2_w, blk1_fc2_b, blk1_bn2_g, blk1_bn2_b),
    ]
    for (fc1_w, fc1_b, bn1_g, bn1_b, mr_w, mr_b, mrbn_g, mrbn_b,
         fc2_w, fc2_b, bn2_g, bn2_b) in blocks:
        f = _fc1_bn(xcur, fc1_w, fc1_b, bn1_g, bn1_b)
        idx = _topk(f.reshape(B, N, C))
        xj = _max_relative(f, idx)
        we = mr_w[:, 0::2]
        wo = mr_w[:, 1::2]
        xcur = _mr_fc2(f, xj, xcur, we, wo, mr_b, mrbn_g, mrbn_b,
                       fc2_w, fc2_b, bn2_g, bn2_b)

    return xcur.reshape(B, 12, 12, 12, C).transpose(0, 4, 1, 2, 3)


# speedup vs baseline: 10.7148x; 10.7148x over previous
import functools

import jax
import jax.numpy as jnp
from jax import lax
from jax.experimental import pallas as pl
from jax.experimental.pallas import tpu as pltpu
from jax.experimental.pallas import tpu_sc as plsc

B = 2
C = 128
N = 1728
M = B * N
K = 9
RT = 216
EPS = 1e-5

NC = 2
NS = 16
NW = NC * NS
GSZ = 8
NG = M // GSZ
GIDX = GSZ * K + GSZ


_INV_M = 0.00028935185400769114


def _rowsum_exact(scr, L):
    acc = scr[0:8, 0:L]
    for i in range(1, M // 8):
        acc = acc + scr[i * 8:(i + 1) * 8, 0:L]
    t = acc[0:4] + acc[4:8]
    t = t[0:2] + t[2:4]
    return t[0:1] + t[1:2]


def _bn_cols(y, g, b, scr):
    L = y.shape[1]
    scr[:, 0:L] = y
    m = _rowsum_exact(scr, L) * _INV_M
    d = y - m
    scr[:, 0:L] = d * d
    v = _rowsum_exact(scr, L) * _INV_M
    return d / jnp.sqrt(v + EPS) * g + b


def _dot_t(a, b):
    return lax.dot_general(a.astype(jnp.bfloat16), b.astype(jnp.bfloat16),
                           (((1,), (1,)), ((), ())),
                           preferred_element_type=jnp.float32)


def _stem_body(p_ref, w_ref, b_ref, o_ref):
    o_ref[...] = _dot_t(p_ref[...], w_ref[...]) + b_ref[...]


def _stem_matmul(patches, w2d, b2d):
    return pl.pallas_call(
        _stem_body,
        grid=(M // RT,),
        in_specs=[
            pl.BlockSpec((RT, 2048), lambda i: (i, 0)),
            pl.BlockSpec((C, 2048), lambda i: (0, 0)),
            pl.BlockSpec((1, C), lambda i: (0, 0)),
        ],
        out_specs=pl.BlockSpec((RT, C), lambda i: (i, 0)),
        out_shape=jax.ShapeDtypeStruct((M, C), jnp.float32),
    )(patches, w2d, b2d)


def _bn_relu_body(x_ref, g_ref, b_ref, o_ref, scr):
    o_ref[...] = jnp.maximum(
        _bn_cols(x_ref[...], g_ref[...], b_ref[...], scr), 0.0)


def _bn_relu(x, g, b):
    return pl.pallas_call(
        _bn_relu_body,
        out_shape=jax.ShapeDtypeStruct(x.shape, jnp.float32),
        scratch_shapes=[pltpu.VMEM((M, C), jnp.float32)],
    )(x, g.reshape(1, -1), b.reshape(1, -1))


def _fc1_body(x_ref, w_ref, b_ref, g_ref, bb_ref, o_ref, scr):
    y = _dot_t(x_ref[...], w_ref[...]) + b_ref[...]
    o_ref[...] = _bn_cols(y, g_ref[...], bb_ref[...], scr)


def _fc1_bn(x, w, b, g, bb):
    return pl.pallas_call(
        _fc1_body,
        out_shape=jax.ShapeDtypeStruct((M, C), jnp.float32),
        scratch_shapes=[pltpu.VMEM((M, C), jnp.float32)],
    )(x, w, b.reshape(1, C), g.reshape(1, C), bb.reshape(1, C))


def _topk_body(fa_ref, ft_ref, o_ref):
    fa = fa_ref[0]
    ft = ft_ref[0]
    b = pl.program_id(0)
    g = _dot_t(ft, fa)
    sqa = jnp.sum(fa * fa, axis=1)[None, :]
    sqt = jnp.sum(ft * ft, axis=1)[:, None]
    dist = sqt - 2.0 * g + sqa
    cols = lax.broadcasted_iota(jnp.int32, (RT, N), 1)
    picks = []
    for _ in range(K):
        m = jnp.min(dist, axis=1, keepdims=True)
        sel = dist == m
        idx = jnp.min(jnp.where(sel, cols, N), axis=1, keepdims=True)
        picks.append(idx)
        dist = jnp.where(cols == idx, jnp.inf, dist)
    o_ref[0] = jnp.concatenate(picks, axis=1) + b * N


def _topk(f3):
    return pl.pallas_call(
        _topk_body,
        grid=(B, N // RT),
        in_specs=[
            pl.BlockSpec((1, N, C), lambda b, r: (b, 0, 0)),
            pl.BlockSpec((1, RT, C), lambda b, r: (b, r, 0)),
        ],
        out_specs=pl.BlockSpec((1, RT, K), lambda b, r: (b, r, 0)),
        out_shape=jax.ShapeDtypeStruct((B, N, K), jnp.int32),
    )(f3, f3)


def _mr_body(cat_ref, xin_ref, wm_ref, mb_ref, mg_ref, mbb_ref,
             w2_ref, b2_ref, g2_ref, b2b_ref, o_ref, scr):
    h = _dot_t(cat_ref[...], wm_ref[...]) + mb_ref[...]
    h = _bn_cols(h, mg_ref[...], mbb_ref[...], scr)
    h = jax.nn.gelu(h)
    out = _dot_t(h, w2_ref[...]) + b2_ref[...]
    out = _bn_cols(out, g2_ref[...], b2b_ref[...], scr)
    o_ref[...] = jnp.maximum(out + xin_ref[...], 0.0)


def _mr_fc2(cat, xin, wm, mb, mg, mbb, w2, b2, g2, b2b):
    return pl.pallas_call(
        _mr_body,
        out_shape=jax.ShapeDtypeStruct((M, C), jnp.float32),
        scratch_shapes=[pltpu.VMEM((M, 2 * C), jnp.float32)],
    )(cat, xin, wm, mb.reshape(1, 2 * C), mg.reshape(1, 2 * C),
      mbb.reshape(1, 2 * C), w2, b2.reshape(1, C), g2.reshape(1, C),
      b2b.reshape(1, C))


def _sc_maxrel_body(f_hbm, idx_hbm, out_hbm, idx_v, rows_v, out_v, sem):
    wid = lax.axis_index("s") * NC + lax.axis_index("c")

    def chunk(ci, carry):
        g = wid + ci * NW
        off = pl.multiple_of(g * GIDX, 8)
        pltpu.sync_copy(idx_hbm.at[pl.ds(off, GIDX)], idx_v)
        pltpu.async_copy(f_hbm.at[idx_v], rows_v, sem).wait()
        for j in range(GSZ):
            for t in range(C // 16):
                s = pl.ds(t * 16, 16)
                acc = rows_v[j * K, s]
                for q in range(1, K):
                    acc = jnp.maximum(acc, rows_v[j * K + q, s])
                out_v[j, s] = acc - rows_v[GSZ * K + j, s]
        pltpu.sync_copy(out_v, out_hbm.at[pl.ds(pl.multiple_of(g * GSZ, 8), GSZ)])
        return carry

    nchunk = jnp.where(wid < NG % NW, NG // NW + 1, NG // NW)
    lax.fori_loop(0, nchunk, chunk, 0)


@functools.cache
def _sc_maxrel():
    return pl.kernel(
        _sc_maxrel_body,
        out_type=jax.ShapeDtypeStruct((M, C), jnp.float32),
        mesh=plsc.VectorSubcoreMesh(core_axis_name="c", subcore_axis_name="s"),
        scratch_types=[
            pltpu.VMEM((GIDX,), jnp.int32),
            pltpu.VMEM((GIDX, C), jnp.float32),
            pltpu.VMEM((GSZ, C), jnp.float32),
            pltpu.SemaphoreType.DMA,
        ],
    )


def _max_relative(f, idx):
    big = jnp.concatenate(
        [idx.reshape(NG, GSZ * K),
         jnp.arange(M, dtype=jnp.int32).reshape(NG, GSZ)], axis=1)
    return _sc_maxrel()(f, big.reshape(NG * GIDX))


def kernel(x, stem_w, stem_b, stem_bng, stem_bnb,
           blk0_fc1_w, blk0_fc1_b, blk0_bn1_g, blk0_bn1_b,
           blk0_mr_w, blk0_mr_b, blk0_mrbn_g, blk0_mrbn_b,
           blk0_fc2_w, blk0_fc2_b, blk0_bn2_g, blk0_bn2_b,
           blk1_fc1_w, blk1_fc1_b, blk1_bn1_g, blk1_bn1_b,
           blk1_mr_w, blk1_mr_b, blk1_mrbn_g, blk1_mrbn_b,
           blk1_fc2_w, blk1_fc2_b, blk1_bn2_g, blk1_bn2_b):
    patches = x.reshape(B, 4, 12, 8, 12, 8, 12, 8)
    patches = patches.transpose(0, 2, 4, 6, 1, 3, 5, 7).reshape(M, 4 * 512)
    y = _stem_matmul(patches, stem_w.reshape(C, 4 * 512), stem_b.reshape(1, C))
    xcur = _bn_relu(y, stem_bng, stem_bnb)

    blocks = [
        (blk0_fc1_w, blk0_fc1_b, blk0_bn1_g, blk0_bn1_b, blk0_mr_w, blk0_mr_b,
         blk0_mrbn_g, blk0_mrbn_b, blk0_fc2_w, blk0_fc2_b, blk0_bn2_g, blk0_bn2_b),
        (blk1_fc1_w, blk1_fc1_b, blk1_bn1_g, blk1_bn1_b, blk1_mr_w, blk1_mr_b,
         blk1_mrbn_g, blk1_mrbn_b, blk1_fc2_w, blk1_fc2_b, blk1_bn2_g, blk1_bn2_b),
    ]
    for (fc1_w, fc1_b, bn1_g, bn1_b, mr_w, mr_b, mrbn_g, mrbn_b,
         fc2_w, fc2_b, bn2_g, bn2_b) in blocks:
        f = _fc1_bn(xcur, fc1_w, fc1_b, bn1_g, bn1_b)
        idx = _topk(f.reshape(B, N, C))
        xj = _max_relative(f, idx)
        cat = jnp.stack([f, xj], axis=2).reshape(M, 2 * C)
        xcur = _mr_fc2(cat, xcur, mr_w, mr_b, mrbn_g, mrbn_b,
                       fc2_w, fc2_b, bn2_g, bn2_b)

    return xcur.reshape(B, 12, 12, 12, C).transpose(0, 4, 1, 2, 3)
